# baseline (speedup 1.0000x reference)
import functools
import math

import jax
import jax.numpy as jnp
from jax import lax
from jax.experimental import pallas as pl
from jax.experimental.pallas import tpu as pltpu
from jax.experimental.pallas import tpu_sc as plsc

_NC = 2
_NS = 16
_NW = _NC * _NS
_LANES = 16
_WIN = 128

_LAMDA = 0.5


_CH = 2048


@functools.lru_cache(maxsize=None)
def _make_prop(n, h, epc):
    cpt = h // _NS
    cps = cpt + 1
    nch = epc // _CH
    assert epc % _CH == 0 and nch % 2 == 0
    ng = _CH // _LANES

    mesh = plsc.VectorSubcoreMesh(core_axis_name="c", subcore_axis_name="s")

    def body(xb_hbm, pk_hbm, nm_hbm, out_hbm, hsl, accs, pkb, nmb, sd0, sd1):
        cid = lax.axis_index("c")
        sid = lax.axis_index("s")
        ebase = cid * epc
        sd = (sd0, sd1)

        def issue(j, b):
            off = ebase + j * _CH
            pltpu.async_copy(pk_hbm.at[pl.ds(off, _CH)], pkb.at[b], sd[b])
            pltpu.async_copy(nm_hbm.at[pl.ds(off, _CH)], nmb.at[b], sd[b])

        def wait(b):
            pltpu.make_async_copy(pk_hbm.at[pl.ds(0, _CH)], pkb.at[b], sd[b]).wait()
            pltpu.make_async_copy(nm_hbm.at[pl.ds(0, _CH)], nmb.at[b], sd[b]).wait()

        issue(0, 0)
        pltpu.sync_copy(xb_hbm.at[sid], hsl)
        zf = jnp.zeros((_LANES,), jnp.float32)

        def zbody(i, cr):
            accs[pl.ds(i * _LANES, _LANES)] = zf
            return cr
        lax.fori_loop(0, n * cps // _LANES, zbody, 0)

        ccs = [jnp.full((_LANES,), c, jnp.int32) for c in range(cpt)]

        def process(b):
            def grp(i, cr):
                base = i * _LANES
                pk16 = pkb[b, pl.ds(base, _LANES)]
                nm16 = nmb[b, pl.ds(base, _LANES)]
                src16 = jnp.bitwise_and(pk16, 0xFFFF)
                dst16 = lax.shift_right_logical(pk16, 16)
                sb = src16 * cps
                db = dst16 * cps
                for c in range(cpt):
                    v = plsc.load_gather(hsl, [sb + ccs[c]])
                    plsc.addupdate_scatter(accs, [db + ccs[c]], v * nm16)
                return cr
            lax.fori_loop(0, ng, grp, 0)

        def jpair(j2, cr):
            j = 2 * j2
            wait(0)
            issue(j + 1, 1)
            process(0)
            wait(1)

            @pl.when(j2 < nch // 2 - 1)
            def _():
                issue(j + 2, 0)
            process(1)
            return cr
        lax.fori_loop(0, nch // 2, jpair, 0)

        pltpu.sync_copy(accs, out_hbm.at[cid, sid])

    return pl.kernel(
        body,
        out_type=jax.ShapeDtypeStruct((2, _NS, n * (h // _NS + 1)), jnp.float32),
        mesh=mesh,
        compiler_params=pltpu.CompilerParams(use_tc_tiling_on_sc=False,
                                             needs_layout_passes=False),
        scratch_types=[
            pltpu.VMEM((n * (h // _NS + 1),), jnp.float32),
            pltpu.VMEM((n * (h // _NS + 1),), jnp.float32),
            pltpu.VMEM((2, _CH), jnp.int32),
            pltpu.VMEM((2, _CH), jnp.float32),
            pltpu.SemaphoreType.DMA,
            pltpu.SemaphoreType.DMA,
        ],
    )


_BLK = 400


def _fc0(features, w, b):
    n, din = features.shape
    hdim = w.shape[1]

    def bdy(x_ref, w_ref, b_ref, o_ref):
        o_ref[...] = jnp.maximum(
            jnp.dot(x_ref[...], w_ref[...], preferred_element_type=jnp.float32)
            + b_ref[...], 0.0)

    return pl.pallas_call(
        bdy,
        grid=(n // _BLK,),
        in_specs=[
            pl.BlockSpec((_BLK, din), lambda i: (i, 0)),
            pl.BlockSpec((din, hdim), lambda i: (0, 0)),
            pl.BlockSpec((1, hdim), lambda i: (0, 0)),
        ],
        out_specs=pl.BlockSpec((_BLK, hdim), lambda i: (i, 0)),
        out_shape=jax.ShapeDtypeStruct((n, hdim), jnp.float32),
    )(features, w, b.reshape(1, hdim))


def _layer(a, h0, pp, prev, w, b, *, beta, tmul, pmul, dorelu):
    n, hdim = h0.shape

    def bdy(a_ref, h0_ref, p0_ref, p1_ref, pv_ref, w_ref, b_ref, o_ref):
        av = a_ref[0]
        tx = tmul * (p0_ref[0] + p1_ref[0]) - pmul * pv_ref[...]
        hi = av * h0_ref[...] + (1.0 - av) * tx
        x = ((1.0 - beta) * hi
             + beta * jnp.dot(hi, w_ref[...], preferred_element_type=jnp.float32)
             + b_ref[...])
        o_ref[...] = jnp.maximum(x, 0.0) if dorelu else x

    return pl.pallas_call(
        bdy,
        grid=(n // _BLK,),
        in_specs=[
            pl.BlockSpec(memory_space=pltpu.SMEM),
            pl.BlockSpec((_BLK, hdim), lambda i: (i, 0)),
            pl.BlockSpec((1, _BLK, hdim), lambda i: (0, i, 0)),
            pl.BlockSpec((1, _BLK, hdim), lambda i: (1, i, 0)),
            pl.BlockSpec((_BLK, hdim), lambda i: (i, 0)),
            pl.BlockSpec((hdim, hdim), lambda i: (0, 0)),
            pl.BlockSpec((1, hdim), lambda i: (0, 0)),
        ],
        out_specs=pl.BlockSpec((_BLK, hdim), lambda i: (i, 0)),
        out_shape=jax.ShapeDtypeStruct((n, hdim), jnp.float32),
    )(a, h0, pp, pp, prev, w, b.reshape(1, hdim))


def _layer0(h0, w, b, *, beta):
    n, hdim = h0.shape

    def bdy(h0_ref, w_ref, b_ref, o_ref):
        hi = h0_ref[...]
        x = ((1.0 - beta) * hi
             + beta * jnp.dot(hi, w_ref[...], preferred_element_type=jnp.float32)
             + b_ref[...])
        o_ref[...] = jnp.maximum(x, 0.0)

    return pl.pallas_call(
        bdy,
        grid=(n // _BLK,),
        in_specs=[
            pl.BlockSpec((_BLK, hdim), lambda i: (i, 0)),
            pl.BlockSpec((hdim, hdim), lambda i: (0, 0)),
            pl.BlockSpec((1, hdim), lambda i: (0, 0)),
        ],
        out_specs=pl.BlockSpec((_BLK, hdim), lambda i: (i, 0)),
        out_shape=jax.ShapeDtypeStruct((n, hdim), jnp.float32),
    )(h0, w, b.reshape(1, hdim))


def _final(x, w, b):
    n, hdim = x.shape
    c = w.shape[1]

    def bdy(x_ref, w_ref, b_ref, o_ref):
        t = jnp.maximum(x_ref[...], 0.0)
        y = (jnp.dot(t, w_ref[...], preferred_element_type=jnp.float32)
             + b_ref[...])
        m = jnp.max(y, axis=1, keepdims=True)
        lse = m + jnp.log(jnp.sum(jnp.exp(y - m), axis=1, keepdims=True))
        o_ref[...] = y - lse

    return pl.pallas_call(
        bdy,
        grid=(n // _BLK,),
        in_specs=[
            pl.BlockSpec((_BLK, hdim), lambda i: (i, 0)),
            pl.BlockSpec((hdim, c), lambda i: (0, 0)),
            pl.BlockSpec((1, c), lambda i: (0, 0)),
        ],
        out_specs=pl.BlockSpec((_BLK, c), lambda i: (i, 0)),
        out_shape=jax.ShapeDtypeStruct((n, c), jnp.float32),
    )(x, w, b.reshape(1, c))


def kernel(features, edge_index, norm_A, W_fc0, b_fc0, conv_W, conv_b,
           W_fc1, b_fc1, alpha_params):
    n = features.shape[0]
    e = norm_A.shape[0]
    hdim = W_fc0.shape[1]
    lnum = conv_W.shape[0] - 1

    epad = -(-e // (4 * _CH)) * (4 * _CH)
    pad = epad - e
    pk = jnp.pad(edge_index[0] + edge_index[1] * 65536, (0, pad))
    nm = jnp.pad(norm_A, (0, pad))
    cpt = hdim // _NS
    cps = cpt + 1
    prop = _make_prop(n, hdim, epad // 2)

    def blocked(x):
        xp = jnp.pad(x.reshape(n, _NS, cpt), ((0, 0), (0, 0), (0, cps - cpt)))
        return xp.transpose(1, 0, 2).reshape(_NS, n * cps)

    def unblocked(o):
        ob = o.reshape(2, _NS, n, cps)[..., :cpt]
        return ob.transpose(0, 2, 1, 3).reshape(2, n, hdim)

    h0 = _fc0(features, W_fc0, b_fc0)
    x = _layer0(h0, conv_W[0], conv_b[0],
                beta=math.log(_LAMDA / 1.0 + 1.0))
    prev = h0
    last = x
    for i in range(1, lnum + 1):
        pp = unblocked(prop(blocked(last), pk, nm))
        a = alpha_params[lnum - i].reshape(1)
        beta = math.log(_LAMDA / (i + 1) + 1.0)
        xi = _layer(a, h0, pp, prev, conv_W[i], conv_b[i],
                    beta=beta, tmul=1.0 if i == 1 else 2.0,
                    pmul=0.0 if i == 1 else 1.0,
                    dorelu=i < lnum - 1)
        prev = last
        last = xi
    return _final(last, W_fc1, b_fc1)


# speedup vs baseline: 1.0216x; 1.0216x over previous
import functools
import math

import jax
import jax.numpy as jnp
from jax import lax
from jax.experimental import pallas as pl
from jax.experimental.pallas import tpu as pltpu
from jax.experimental.pallas import tpu_sc as plsc

_NC = 2
_NS = 16
_NW = _NC * _NS
_LANES = 16
_WIN = 128

_LAMDA = 0.5


_CH = 2048


@functools.lru_cache(maxsize=None)
def _make_prop(n, h, epc):
    cpt = h // _NS
    cps = cpt + 1
    nch = epc // _CH
    assert epc % _CH == 0 and nch % 2 == 0
    ng = _CH // _LANES

    mesh = plsc.VectorSubcoreMesh(core_axis_name="c", subcore_axis_name="s")

    def body(xb_hbm, pk_hbm, nm_hbm, out_hbm, hsl, accs, pkb, nmb, sd0, sd1):
        cid = lax.axis_index("c")
        sid = lax.axis_index("s")
        ebase = cid * epc
        sd = (sd0, sd1)

        def issue(j, b):
            off = ebase + j * _CH
            pltpu.async_copy(pk_hbm.at[pl.ds(off, _CH)], pkb.at[b], sd[b])
            pltpu.async_copy(nm_hbm.at[pl.ds(off, _CH)], nmb.at[b], sd[b])

        def wait(b):
            pltpu.make_async_copy(pk_hbm.at[pl.ds(0, _CH)], pkb.at[b], sd[b]).wait()
            pltpu.make_async_copy(nm_hbm.at[pl.ds(0, _CH)], nmb.at[b], sd[b]).wait()

        issue(0, 0)
        pltpu.sync_copy(xb_hbm.at[sid], hsl)
        zf = jnp.zeros((_LANES,), jnp.float32)

        def zbody(i, cr):
            accs[pl.ds(i * _LANES, _LANES)] = zf
            return cr
        lax.fori_loop(0, n * cps // _LANES, zbody, 0)

        ccs = [jnp.full((_LANES,), c, jnp.int32) for c in range(cpt)]

        def process(b):
            def grp(i, cr):
                base = i * _LANES
                pk16 = pkb[b, pl.ds(base, _LANES)]
                nm16 = nmb[b, pl.ds(base, _LANES)]
                src16 = jnp.bitwise_and(pk16, 0xFFFF)
                dst16 = lax.shift_right_logical(pk16, 16)
                sb = src16 * cps
                db = dst16 * cps
                for c in range(cpt):
                    v = plsc.load_gather(hsl, [sb + ccs[c]])
                    plsc.addupdate_scatter(accs, [db + ccs[c]], v * nm16)
                return cr
            lax.fori_loop(0, ng, grp, 0, unroll=8)

        def jpair(j2, cr):
            j = 2 * j2
            wait(0)
            issue(j + 1, 1)
            process(0)
            wait(1)

            @pl.when(j2 < nch // 2 - 1)
            def _():
                issue(j + 2, 0)
            process(1)
            return cr
        lax.fori_loop(0, nch // 2, jpair, 0)

        pltpu.sync_copy(accs, out_hbm.at[cid, sid])

    return pl.kernel(
        body,
        out_type=jax.ShapeDtypeStruct((2, _NS, n * (h // _NS + 1)), jnp.float32),
        mesh=mesh,
        compiler_params=pltpu.CompilerParams(use_tc_tiling_on_sc=False,
                                             needs_layout_passes=False),
        scratch_types=[
            pltpu.VMEM((n * (h // _NS + 1),), jnp.float32),
            pltpu.VMEM((n * (h // _NS + 1),), jnp.float32),
            pltpu.VMEM((2, _CH), jnp.int32),
            pltpu.VMEM((2, _CH), jnp.float32),
            pltpu.SemaphoreType.DMA,
            pltpu.SemaphoreType.DMA,
        ],
    )


_BLK = 400


def _fc0(features, w, b):
    n, din = features.shape
    hdim = w.shape[1]

    def bdy(x_ref, w_ref, b_ref, o_ref):
        o_ref[...] = jnp.maximum(
            jnp.dot(x_ref[...], w_ref[...], preferred_element_type=jnp.float32)
            + b_ref[...], 0.0)

    return pl.pallas_call(
        bdy,
        grid=(n // _BLK,),
        in_specs=[
            pl.BlockSpec((_BLK, din), lambda i: (i, 0)),
            pl.BlockSpec((din, hdim), lambda i: (0, 0)),
            pl.BlockSpec((1, hdim), lambda i: (0, 0)),
        ],
        out_specs=pl.BlockSpec((_BLK, hdim), lambda i: (i, 0)),
        out_shape=jax.ShapeDtypeStruct((n, hdim), jnp.float32),
    )(features, w, b.reshape(1, hdim))


def _layer(a, h0, pp, prev, w, b, *, beta, tmul, pmul, dorelu):
    n, hdim = h0.shape

    def bdy(a_ref, h0_ref, p0_ref, p1_ref, pv_ref, w_ref, b_ref, o_ref):
        av = a_ref[0]
        tx = tmul * (p0_ref[0] + p1_ref[0]) - pmul * pv_ref[...]
        hi = av * h0_ref[...] + (1.0 - av) * tx
        x = ((1.0 - beta) * hi
             + beta * jnp.dot(hi, w_ref[...], preferred_element_type=jnp.float32)
             + b_ref[...])
        o_ref[...] = jnp.maximum(x, 0.0) if dorelu else x

    return pl.pallas_call(
        bdy,
        grid=(n // _BLK,),
        in_specs=[
            pl.BlockSpec(memory_space=pltpu.SMEM),
            pl.BlockSpec((_BLK, hdim), lambda i: (i, 0)),
            pl.BlockSpec((1, _BLK, hdim), lambda i: (0, i, 0)),
            pl.BlockSpec((1, _BLK, hdim), lambda i: (1, i, 0)),
            pl.BlockSpec((_BLK, hdim), lambda i: (i, 0)),
            pl.BlockSpec((hdim, hdim), lambda i: (0, 0)),
            pl.BlockSpec((1, hdim), lambda i: (0, 0)),
        ],
        out_specs=pl.BlockSpec((_BLK, hdim), lambda i: (i, 0)),
        out_shape=jax.ShapeDtypeStruct((n, hdim), jnp.float32),
    )(a, h0, pp, pp, prev, w, b.reshape(1, hdim))


def _layer0(h0, w, b, *, beta):
    n, hdim = h0.shape

    def bdy(h0_ref, w_ref, b_ref, o_ref):
        hi = h0_ref[...]
        x = ((1.0 - beta) * hi
             + beta * jnp.dot(hi, w_ref[...], preferred_element_type=jnp.float32)
             + b_ref[...])
        o_ref[...] = jnp.maximum(x, 0.0)

    return pl.pallas_call(
        bdy,
        grid=(n // _BLK,),
        in_specs=[
            pl.BlockSpec((_BLK, hdim), lambda i: (i, 0)),
            pl.BlockSpec((hdim, hdim), lambda i: (0, 0)),
            pl.BlockSpec((1, hdim), lambda i: (0, 0)),
        ],
        out_specs=pl.BlockSpec((_BLK, hdim), lambda i: (i, 0)),
        out_shape=jax.ShapeDtypeStruct((n, hdim), jnp.float32),
    )(h0, w, b.reshape(1, hdim))


def _final(x, w, b):
    n, hdim = x.shape
    c = w.shape[1]

    def bdy(x_ref, w_ref, b_ref, o_ref):
        t = jnp.maximum(x_ref[...], 0.0)
        y = (jnp.dot(t, w_ref[...], preferred_element_type=jnp.float32)
             + b_ref[...])
        m = jnp.max(y, axis=1, keepdims=True)
        lse = m + jnp.log(jnp.sum(jnp.exp(y - m), axis=1, keepdims=True))
        o_ref[...] = y - lse

    return pl.pallas_call(
        bdy,
        grid=(n // _BLK,),
        in_specs=[
            pl.BlockSpec((_BLK, hdim), lambda i: (i, 0)),
            pl.BlockSpec((hdim, c), lambda i: (0, 0)),
            pl.BlockSpec((1, c), lambda i: (0, 0)),
        ],
        out_specs=pl.BlockSpec((_BLK, c), lambda i: (i, 0)),
        out_shape=jax.ShapeDtypeStruct((n, c), jnp.float32),
    )(x, w, b.reshape(1, c))


def kernel(features, edge_index, norm_A, W_fc0, b_fc0, conv_W, conv_b,
           W_fc1, b_fc1, alpha_params):
    n = features.shape[0]
    e = norm_A.shape[0]
    hdim = W_fc0.shape[1]
    lnum = conv_W.shape[0] - 1

    epad = -(-e // (4 * _CH)) * (4 * _CH)
    pad = epad - e
    pk = jnp.pad(edge_index[0] + edge_index[1] * 65536, (0, pad))
    nm = jnp.pad(norm_A, (0, pad))
    cpt = hdim // _NS
    cps = cpt + 1
    prop = _make_prop(n, hdim, epad // 2)

    def blocked(x):
        xp = jnp.pad(x.reshape(n, _NS, cpt), ((0, 0), (0, 0), (0, cps - cpt)))
        return xp.transpose(1, 0, 2).reshape(_NS, n * cps)

    def unblocked(o):
        ob = o.reshape(2, _NS, n, cps)[..., :cpt]
        return ob.transpose(0, 2, 1, 3).reshape(2, n, hdim)

    h0 = _fc0(features, W_fc0, b_fc0)
    x = _layer0(h0, conv_W[0], conv_b[0],
                beta=math.log(_LAMDA / 1.0 + 1.0))
    prev = h0
    last = x
    for i in range(1, lnum + 1):
        pp = unblocked(prop(blocked(last), pk, nm))
        a = alpha_params[lnum - i].reshape(1)
        beta = math.log(_LAMDA / (i + 1) + 1.0)
        xi = _layer(a, h0, pp, prev, conv_W[i], conv_b[i],
                    beta=beta, tmul=1.0 if i == 1 else 2.0,
                    pmul=0.0 if i == 1 else 1.0,
                    dorelu=i < lnum - 1)
        prev = last
        last = xi
    return _final(last, W_fc1, b_fc1)


# speedup vs baseline: 2.7415x; 2.6837x over previous
import functools
import math

import jax
import jax.numpy as jnp
from jax import lax
from jax.experimental import pallas as pl
from jax.experimental.pallas import tpu as pltpu
from jax.experimental.pallas import tpu_sc as plsc

_NC = 2
_NS = 16
_NW = _NC * _NS
_LANES = 16
_WIN = 128

_LAMDA = 0.5


_NBUF = 3


@functools.lru_cache(maxsize=None)
def _make_prop(n, wpw, h):
    assert wpw % _NBUF == 0 and wpw >= 2 * _NBUF
    rps = (-(-n // _NS) + 7) // 8 * 8
    npad = rps * _NS
    ncol = h // _LANES
    assert n % _NS == 0
    hps = n // _NS
    nk3 = wpw // _NBUF

    mesh = plsc.VectorSubcoreMesh(core_axis_name="c", subcore_axis_name="s")

    def body(h_hbm, pk_hbm, norm_hbm, zer_hbm, out_hbm,
             acc, hsp, pkb, srcw, dstw, nbuf, rows,
             sg0, sr0, sr1, sr2, ss0, ss1, ss2):
        cid = lax.axis_index("c")
        sid = lax.axis_index("s")
        wid = sid * _NC + cid
        row0 = wid * wpw
        sr = (sr0, sr1, sr2)
        ss = (ss0, ss1, ss2)

        def unpack(k, q):
            for j in range(_WIN // _LANES):
                sl = pl.ds(j * _LANES, _LANES)
                pkv = pkb[k, sl]
                srcw[q, sl] = jnp.bitwise_and(pkv, 0xFFFF)
                dstw[q, sl] = lax.shift_right_logical(pkv, 16)

        def issue_gather(k, p):
            pltpu.async_copy(hsp.at[srcw.at[p]], rows.at[p], sr[p])

        def wait_gather(p):
            pltpu.make_async_copy(hsp.at[pl.ds(0, _WIN)], rows.at[p], sr[p]).wait()

        def issue_scatter(p):
            pltpu.async_copy(rows.at[p], acc.at[dstw.at[p]], ss[p], add=True)

        def wait_scatter(p):
            pltpu.make_async_copy(h_hbm.at[pl.ds(0, _WIN)], rows.at[p], ss[p]).wait()

        cp_p = pltpu.async_copy(pk_hbm.at[pl.ds(row0, wpw)], pkb, sg0)
        cp_n = pltpu.async_copy(norm_hbm.at[pl.ds(row0, wpw)], nbuf, sg0)

        zf = jnp.zeros((_LANES,), jnp.float32)

        def zrow(r, c):
            for cc in range(ncol):
                rows[_NBUF - 1, r, pl.ds(cc * _LANES, _LANES)] = zf
            return c
        lax.fori_loop(0, _WIN, zrow, 0)
        for cc in range(_WIN // _LANES):
            dstw[_NBUF - 1, pl.ds(cc * _LANES, _LANES)] = (
                lax.iota(jnp.int32, _LANES) + cc * _LANES)

        pltpu.sync_copy(zer_hbm, acc.at[pl.ds(sid * rps, rps)])
        pltpu.sync_copy(h_hbm.at[pl.ds(sid * hps, hps)],
                        hsp.at[pl.ds(sid * hps, hps)])
        plsc.subcore_barrier()

        issue_scatter(_NBUF - 1)

        cp_p.wait()
        cp_n.wait()
        for k in range(_NBUF - 1):
            unpack(k, k)
            issue_gather(k, k)

        def scale(k, p):
            def gbody(g16, c2):
                nv16 = nbuf[k, pl.ds(g16 * _LANES, _LANES)]
                for l in range(_LANES):
                    vb = jnp.full((_LANES,), nv16[l], jnp.float32)
                    ei = g16 * _LANES + l
                    for cc in range(ncol):
                        sl = pl.ds(cc * _LANES, _LANES)
                        rows[p, ei, sl] = rows[p, ei, sl] * vb
                return c2
            lax.fori_loop(0, _WIN // _LANES, gbody, 0)

        def window(k3, sub):
            k = _NBUF * k3 + sub
            p = sub
            q = (sub + _NBUF - 1) % _NBUF
            wait_gather(p)

            def ahead():
                wait_scatter(q)
                unpack(k + _NBUF - 1, q)
                issue_gather(k + _NBUF - 1, q)

            if sub == 0:
                ahead()
            else:
                pl.when(k3 < nk3 - 1)(ahead)

            scale(k, p)
            issue_scatter(p)

        def k3body(k3, c):
            for sub in range(_NBUF):
                window(k3, sub)
            return c
        lax.fori_loop(0, nk3, k3body, 0)

        for p in range(_NBUF):
            wait_scatter(p)
        plsc.subcore_barrier()
        pltpu.sync_copy(acc.at[pl.ds(sid * rps, rps)],
                        out_hbm.at[cid, pl.ds(sid * rps, rps)])

    return pl.kernel(
        body,
        out_type=jax.ShapeDtypeStruct((2, npad, h), jnp.float32),
        mesh=mesh,
        compiler_params=pltpu.CompilerParams(use_tc_tiling_on_sc=False),
        scratch_types=[
            pltpu.VMEM_SHARED((npad, h), jnp.float32),
            pltpu.VMEM_SHARED((n, h), jnp.float32),
            pltpu.VMEM((wpw, _WIN), jnp.int32),
            pltpu.VMEM((_NBUF, _WIN), jnp.int32),
            pltpu.VMEM((_NBUF, _WIN), jnp.int32),
            pltpu.VMEM((wpw, _WIN), jnp.float32),
            pltpu.VMEM((_NBUF, _WIN, h), jnp.float32),
            pltpu.SemaphoreType.DMA,
            pltpu.SemaphoreType.DMA,
            pltpu.SemaphoreType.DMA,
            pltpu.SemaphoreType.DMA,
            pltpu.SemaphoreType.DMA,
            pltpu.SemaphoreType.DMA,
            pltpu.SemaphoreType.DMA,
        ],
    )


_BLK = 400


def _fc0(features, w, b):
    n, din = features.shape
    hdim = w.shape[1]

    def bdy(x_ref, w_ref, b_ref, o_ref):
        o_ref[...] = jnp.maximum(
            jnp.dot(x_ref[...], w_ref[...], preferred_element_type=jnp.float32)
            + b_ref[...], 0.0)

    return pl.pallas_call(
        bdy,
        grid=(n // _BLK,),
        in_specs=[
            pl.BlockSpec((_BLK, din), lambda i: (i, 0)),
            pl.BlockSpec((din, hdim), lambda i: (0, 0)),
            pl.BlockSpec((1, hdim), lambda i: (0, 0)),
        ],
        out_specs=pl.BlockSpec((_BLK, hdim), lambda i: (i, 0)),
        out_shape=jax.ShapeDtypeStruct((n, hdim), jnp.float32),
    )(features, w, b.reshape(1, hdim))


def _layer(a, h0, pp, prev, w, b, *, beta, tmul, pmul, dorelu):
    n, hdim = h0.shape

    def bdy(a_ref, h0_ref, p0_ref, p1_ref, pv_ref, w_ref, b_ref, o_ref):
        av = a_ref[0]
        tx = tmul * (p0_ref[0] + p1_ref[0]) - pmul * pv_ref[...]
        hi = av * h0_ref[...] + (1.0 - av) * tx
        x = ((1.0 - beta) * hi
             + beta * jnp.dot(hi, w_ref[...], preferred_element_type=jnp.float32)
             + b_ref[...])
        o_ref[...] = jnp.maximum(x, 0.0) if dorelu else x

    return pl.pallas_call(
        bdy,
        grid=(n // _BLK,),
        in_specs=[
            pl.BlockSpec(memory_space=pltpu.SMEM),
            pl.BlockSpec((_BLK, hdim), lambda i: (i, 0)),
            pl.BlockSpec((1, _BLK, hdim), lambda i: (0, i, 0)),
            pl.BlockSpec((1, _BLK, hdim), lambda i: (1, i, 0)),
            pl.BlockSpec((_BLK, hdim), lambda i: (i, 0)),
            pl.BlockSpec((hdim, hdim), lambda i: (0, 0)),
            pl.BlockSpec((1, hdim), lambda i: (0, 0)),
        ],
        out_specs=pl.BlockSpec((_BLK, hdim), lambda i: (i, 0)),
        out_shape=jax.ShapeDtypeStruct((n, hdim), jnp.float32),
    )(a, h0, pp, pp, prev, w, b.reshape(1, hdim))


def _layer0(h0, w, b, *, beta):
    n, hdim = h0.shape

    def bdy(h0_ref, w_ref, b_ref, o_ref):
        hi = h0_ref[...]
        x = ((1.0 - beta) * hi
             + beta * jnp.dot(hi, w_ref[...], preferred_element_type=jnp.float32)
             + b_ref[...])
        o_ref[...] = jnp.maximum(x, 0.0)

    return pl.pallas_call(
        bdy,
        grid=(n // _BLK,),
        in_specs=[
            pl.BlockSpec((_BLK, hdim), lambda i: (i, 0)),
            pl.BlockSpec((hdim, hdim), lambda i: (0, 0)),
            pl.BlockSpec((1, hdim), lambda i: (0, 0)),
        ],
        out_specs=pl.BlockSpec((_BLK, hdim), lambda i: (i, 0)),
        out_shape=jax.ShapeDtypeStruct((n, hdim), jnp.float32),
    )(h0, w, b.reshape(1, hdim))


def _final(x, w, b):
    n, hdim = x.shape
    c = w.shape[1]

    def bdy(x_ref, w_ref, b_ref, o_ref):
        t = jnp.maximum(x_ref[...], 0.0)
        y = (jnp.dot(t, w_ref[...], preferred_element_type=jnp.float32)
             + b_ref[...])
        m = jnp.max(y, axis=1, keepdims=True)
        lse = m + jnp.log(jnp.sum(jnp.exp(y - m), axis=1, keepdims=True))
        o_ref[...] = y - lse

    return pl.pallas_call(
        bdy,
        grid=(n // _BLK,),
        in_specs=[
            pl.BlockSpec((_BLK, hdim), lambda i: (i, 0)),
            pl.BlockSpec((hdim, c), lambda i: (0, 0)),
            pl.BlockSpec((1, c), lambda i: (0, 0)),
        ],
        out_specs=pl.BlockSpec((_BLK, c), lambda i: (i, 0)),
        out_shape=jax.ShapeDtypeStruct((n, c), jnp.float32),
    )(x, w, b.reshape(1, c))


def kernel(features, edge_index, norm_A, W_fc0, b_fc0, conv_W, conv_b,
           W_fc1, b_fc1, alpha_params):
    n = features.shape[0]
    e = norm_A.shape[0]
    hdim = W_fc0.shape[1]
    lnum = conv_W.shape[0] - 1

    wpw = -(-e // (_WIN * _NW))
    wpw = (wpw + _NBUF - 1) // _NBUF * _NBUF
    wpw = max(wpw, 2 * _NBUF)
    epad = wpw * _NW * _WIN
    pad = epad - e
    pk2 = jnp.pad(edge_index[0] + edge_index[1] * 65536,
                  (0, pad)).reshape(epad // _WIN, _WIN)
    norm2 = jnp.pad(norm_A, (0, pad)).reshape(epad // _WIN, _WIN)
    zer = jnp.zeros(((-(-n // _NS) + 7) // 8 * 8, hdim), jnp.float32)
    prop = _make_prop(n, wpw, hdim)

    h0 = _fc0(features, W_fc0, b_fc0)
    x = _layer0(h0, conv_W[0], conv_b[0],
                beta=math.log(_LAMDA / 1.0 + 1.0))
    prev = h0
    last = x
    for i in range(1, lnum + 1):
        pp = prop(last, pk2, norm2, zer)
        a = alpha_params[lnum - i].reshape(1)
        beta = math.log(_LAMDA / (i + 1) + 1.0)
        xi = _layer(a, h0, pp, prev, conv_W[i], conv_b[i],
                    beta=beta, tmul=1.0 if i == 1 else 2.0,
                    pmul=0.0 if i == 1 else 1.0,
                    dorelu=i < lnum - 1)
        prev = last
        last = xi
    return _final(last, W_fc1, b_fc1)
